# baseline (speedup 1.0000x reference)
import jax
import jax.numpy as jnp
from jax import lax
from jax.experimental import pallas as pl
from jax.experimental.pallas import tpu as pltpu
from jax.experimental.pallas import tpu_sc as plsc

N = 10000
E = 320000
D = 128
BN_EPS = 1e-5

NC = 2
NS = 16
NW = NC * NS

CHUNK = 112
EPW = -(-E // (NW * CHUNK))
E_PAD = NW * EPW * CHUNK
N_SUB = 624
N_REM = N - NS * N_SUB
N_ACC = N + 8


def _sc_agg_body(x_hbm, row_hbm, col_hbm, z_hbm, out_hbm,
                 ridx, cidx, buf0, agg_sh,
                 sem_r, sem_c):
    c = lax.axis_index("c")
    s = lax.axis_index("s")
    w = c * NS + s

    cp_r = pltpu.make_async_copy(row_hbm.at[w], ridx, sem_r)
    cp_c = pltpu.make_async_copy(col_hbm.at[w], cidx, sem_c)
    cp_r.start()
    cp_c.start()

    pltpu.sync_copy(z_hbm, agg_sh.at[pl.ds(s * N_SUB, N_SUB)])

    @pl.when(s == 0)
    def _():
        pltpu.sync_copy(z_hbm.at[pl.ds(0, N_ACC - NS * N_SUB)],
                        agg_sh.at[pl.ds(NS * N_SUB, N_ACC - NS * N_SUB)])

    cp_r.wait()
    cp_c.wait()
    plsc.subcore_barrier()

    @pl.loop(0, EPW)
    def _(j):
        pltpu.sync_copy(x_hbm.at[ridx.at[j]], buf0)
        pltpu.sync_copy(buf0, agg_sh.at[cidx.at[j]], add=True)

    plsc.subcore_barrier()
    pltpu.sync_copy(agg_sh.at[pl.ds(s * N_SUB, N_SUB)],
                    out_hbm.at[c, pl.ds(s * N_SUB, N_SUB)])

    @pl.when(s == 0)
    def _():
        pltpu.sync_copy(agg_sh.at[pl.ds(NS * N_SUB, N_REM)],
                        out_hbm.at[c, pl.ds(NS * N_SUB, N_REM)])


def _sc_aggregate(x, row3, col3, zeros_block):
    mesh = plsc.VectorSubcoreMesh(core_axis_name="c", subcore_axis_name="s",
                                  num_cores=NC, num_subcores=NS)
    kern = pl.kernel(
        _sc_agg_body,
        out_type=jax.ShapeDtypeStruct((NC, N, D), jnp.float32),
        mesh=mesh,
        scratch_types=[
            pltpu.VMEM((EPW, CHUNK), jnp.int32),
            pltpu.VMEM((EPW, CHUNK), jnp.int32),
            pltpu.VMEM((CHUNK, D), jnp.float32),
            pltpu.VMEM_SHARED((N_ACC, D), jnp.float32),
            pltpu.SemaphoreType.DMA,
            pltpu.SemaphoreType.DMA,
        ],
    )
    return kern(x, row3, col3, zeros_block)


def _mlp_body(eps_ref, x_ref, a0_ref, a1_ref, w1_ref, b1_ref, g1_ref,
              be1_ref, w2_ref, b2_ref, g2_ref, be2_ref, o_ref):
    h = x_ref[...] * (1.0 + eps_ref[0]) + a0_ref[...] + a1_ref[...]
    h = jnp.dot(h, w1_ref[...], preferred_element_type=jnp.float32)
    h = h + b1_ref[...]
    m = jnp.mean(h, axis=0, keepdims=True)
    hc = h - m
    v = jnp.mean(hc * hc, axis=0, keepdims=True)
    h = hc * lax.rsqrt(v + BN_EPS) * g1_ref[...] + be1_ref[...]
    h = jnp.maximum(h, 0.0)
    h = jnp.dot(h, w2_ref[...], preferred_element_type=jnp.float32)
    h = h + b2_ref[...]
    m2 = jnp.mean(h, axis=0, keepdims=True)
    hc2 = h - m2
    v2 = jnp.mean(hc2 * hc2, axis=0, keepdims=True)
    o_ref[...] = hc2 * lax.rsqrt(v2 + BN_EPS) * g2_ref[...] + be2_ref[...]


def _mlp(eps, x, a0, a1, W1, b1, g1, be1, W2, b2, g2, be2):
    smem_spec = pl.BlockSpec(memory_space=pltpu.SMEM)
    vmem_spec = pl.BlockSpec(memory_space=pltpu.VMEM)
    return pl.pallas_call(
        _mlp_body,
        out_shape=jax.ShapeDtypeStruct((N, D), jnp.float32),
        in_specs=[smem_spec] + [vmem_spec] * 11,
        out_specs=vmem_spec,
    )(eps, x, a0, a1, W1, b1, g1, be1, W2, b2, g2, be2)


@jax.jit
def kernel(x, edge_index, W1, b1, g1, be1, W2, b2, g2, be2, eps):
    pad = jnp.stack([jnp.zeros((E_PAD - E,), jnp.int32),
                     jnp.full((E_PAD - E,), N, jnp.int32)])
    ei = jnp.concatenate([edge_index, pad], axis=1)
    row3 = ei[0].reshape(NW, EPW, CHUNK)
    col3 = ei[1].reshape(NW, EPW, CHUNK)
    zeros_block = jnp.zeros((N_SUB, D), jnp.float32)
    parts = _sc_aggregate(x, row3, col3, zeros_block)
    return _mlp(eps, x, parts[0], parts[1],
                W1, b1.reshape(1, D), g1.reshape(1, D), be1.reshape(1, D),
                W2, b2.reshape(1, D), g2.reshape(1, D), be2.reshape(1, D))


# speedup vs baseline: 1.5119x; 1.5119x over previous
import jax
import jax.numpy as jnp
from jax import lax
from jax.experimental import pallas as pl
from jax.experimental.pallas import tpu as pltpu
from jax.experimental.pallas import tpu_sc as plsc

N = 10000
E = 320000
D = 128
BN_EPS = 1e-5

NC = 2
NS = 16
NW = NC * NS

CHUNK = 125
EPW = E // (NW * CHUNK)
N_SUB = 624
N_REM = N - NS * N_SUB
N_ACC = N


def _sc_agg_body(x_hbm, row_hbm, col_hbm, z_hbm, out_hbm,
                 ridx, cidx, buf0, agg_sh,
                 sem_r, sem_c):
    c = lax.axis_index("c")
    s = lax.axis_index("s")
    w = c * NS + s

    cp_r = pltpu.make_async_copy(row_hbm.at[w], ridx, sem_r)
    cp_c = pltpu.make_async_copy(col_hbm.at[w], cidx, sem_c)
    cp_r.start()
    cp_c.start()

    pltpu.sync_copy(z_hbm, agg_sh.at[pl.ds(s * N_SUB, N_SUB)])

    @pl.when(s == 0)
    def _():
        pltpu.sync_copy(z_hbm.at[pl.ds(0, N_ACC - NS * N_SUB)],
                        agg_sh.at[pl.ds(NS * N_SUB, N_ACC - NS * N_SUB)])

    cp_r.wait()
    cp_c.wait()
    plsc.subcore_barrier()

    @pl.loop(0, EPW)
    def _(j):
        pltpu.sync_copy(x_hbm.at[ridx.at[j]], buf0)
        pltpu.sync_copy(buf0, agg_sh.at[cidx.at[j]], add=True)

    plsc.subcore_barrier()
    pltpu.sync_copy(agg_sh.at[pl.ds(s * N_SUB, N_SUB)],
                    out_hbm.at[c, pl.ds(s * N_SUB, N_SUB)])

    @pl.when(s == 0)
    def _():
        pltpu.sync_copy(agg_sh.at[pl.ds(NS * N_SUB, N_REM)],
                        out_hbm.at[c, pl.ds(NS * N_SUB, N_REM)])


def _sc_aggregate(x, row3, col3, zeros_block):
    mesh = plsc.VectorSubcoreMesh(core_axis_name="c", subcore_axis_name="s",
                                  num_cores=NC, num_subcores=NS)
    kern = pl.kernel(
        _sc_agg_body,
        out_type=jax.ShapeDtypeStruct((NC, N, D), jnp.float32),
        mesh=mesh,
        scratch_types=[
            pltpu.VMEM((EPW, CHUNK), jnp.int32),
            pltpu.VMEM((EPW, CHUNK), jnp.int32),
            pltpu.VMEM((CHUNK, D), jnp.float32),
            pltpu.VMEM_SHARED((N_ACC, D), jnp.float32),
            pltpu.SemaphoreType.DMA,
            pltpu.SemaphoreType.DMA,
        ],
    )
    return kern(x, row3, col3, zeros_block)


def _mlp_body(eps_ref, x_ref, a0_ref, a1_ref, w1_ref, b1_ref, g1_ref,
              be1_ref, w2_ref, b2_ref, g2_ref, be2_ref, o_ref):
    h = x_ref[...] * (1.0 + eps_ref[0]) + a0_ref[...] + a1_ref[...]
    h = jnp.dot(h, w1_ref[...], preferred_element_type=jnp.float32)
    h = h + b1_ref[...]
    m = jnp.mean(h, axis=0, keepdims=True)
    hc = h - m
    v = jnp.mean(hc * hc, axis=0, keepdims=True)
    h = hc * lax.rsqrt(v + BN_EPS) * g1_ref[...] + be1_ref[...]
    h = jnp.maximum(h, 0.0)
    h = jnp.dot(h, w2_ref[...], preferred_element_type=jnp.float32)
    h = h + b2_ref[...]
    m2 = jnp.mean(h, axis=0, keepdims=True)
    hc2 = h - m2
    v2 = jnp.mean(hc2 * hc2, axis=0, keepdims=True)
    o_ref[...] = hc2 * lax.rsqrt(v2 + BN_EPS) * g2_ref[...] + be2_ref[...]


def _mlp(eps, x, a0, a1, W1, b1, g1, be1, W2, b2, g2, be2):
    smem_spec = pl.BlockSpec(memory_space=pltpu.SMEM)
    vmem_spec = pl.BlockSpec(memory_space=pltpu.VMEM)
    return pl.pallas_call(
        _mlp_body,
        out_shape=jax.ShapeDtypeStruct((N, D), jnp.float32),
        in_specs=[smem_spec] + [vmem_spec] * 11,
        out_specs=vmem_spec,
    )(eps, x, a0, a1, W1, b1, g1, be1, W2, b2, g2, be2)


@jax.jit
def kernel(x, edge_index, W1, b1, g1, be1, W2, b2, g2, be2, eps):
    row3 = edge_index[0].reshape(NW, EPW, CHUNK)
    col3 = edge_index[1].reshape(NW, EPW, CHUNK)
    zeros_block = jnp.zeros((N_SUB, D), jnp.float32)
    parts = _sc_aggregate(x, row3, col3, zeros_block)
    return _mlp(eps, x, parts[0], parts[1],
                W1, b1.reshape(1, D), g1.reshape(1, D), be1.reshape(1, D),
                W2, b2.reshape(1, D), g2.reshape(1, D), be2.reshape(1, D))


# speedup vs baseline: 2.1260x; 1.4061x over previous
import jax
import jax.numpy as jnp
from jax import lax
from jax.experimental import pallas as pl
from jax.experimental.pallas import tpu as pltpu
from jax.experimental.pallas import tpu_sc as plsc

N = 10000
E = 320000
D = 128
BN_EPS = 1e-5

NC = 2
NS = 16
NW = NC * NS

CHUNK = 125
EPW = E // (NW * CHUNK)
N_SUB = 624
N_REM = N - NS * N_SUB
N_ACC = N


NBANK = 8
NBANKS = EPW // NBANK


def _sc_agg_body(x_hbm, row_hbm, col_hbm, z_hbm, out_hbm,
                 ridx, cidx, buf0, buf1, agg_sh,
                 sem_r, sem_c, sem0, sem1):
    c = lax.axis_index("c")
    s = lax.axis_index("s")
    w = c * NS + s

    cp_r = pltpu.make_async_copy(row_hbm.at[w], ridx, sem_r)
    cp_r.start()
    pltpu.sync_copy(col_hbm.at[w, pl.ds(0, NBANK)], cidx.at[pl.ds(0, NBANK)])
    pltpu.make_async_copy(col_hbm.at[w, pl.ds(NBANK, NBANK)],
                          cidx.at[pl.ds(NBANK, NBANK)], sem_c).start()

    pltpu.sync_copy(z_hbm, agg_sh.at[pl.ds(s * N_SUB, N_SUB)])

    @pl.when(s == 0)
    def _():
        pltpu.sync_copy(z_hbm.at[pl.ds(0, N_REM)],
                        agg_sh.at[pl.ds(NS * N_SUB, N_REM)])

    cp_r.wait()
    plsc.subcore_barrier()

    def gather_start(j, buf, sem):
        pltpu.make_async_copy(x_hbm.at[ridx.at[j]], buf, sem).start()

    def gather_wait(j, buf, sem):
        pltpu.make_async_copy(x_hbm.at[ridx.at[j]], buf, sem).wait()

    bufs = (buf0, buf1)
    sems = (sem0, sem1)
    gather_start(0, buf0, sem0)
    gather_start(1, buf1, sem1)

    @pl.loop(0, NBANKS)
    def _(b):
        pb = lax.rem(b, 2)

        @pl.when(b > 0)
        def _():
            pltpu.make_async_copy(col_hbm.at[w, pl.ds(b * NBANK, NBANK)],
                                  cidx.at[pl.ds(pb * NBANK, NBANK)],
                                  sem_c).wait()

        @pl.when(b < NBANKS - 1)
        def _():
            pltpu.make_async_copy(
                col_hbm.at[w, pl.ds((b + 1) * NBANK, NBANK)],
                cidx.at[pl.ds((1 - pb) * NBANK, NBANK)], sem_c).start()

        for k in range(NBANK):
            j = b * NBANK + k
            p = k & 1
            gather_wait(j, bufs[p], sems[p])
            pltpu.sync_copy(bufs[p], agg_sh.at[cidx.at[pb * NBANK + k]],
                            add=True)

            @pl.when(j + 2 < EPW)
            def _():
                gather_start(j + 2, bufs[p], sems[p])

    plsc.subcore_barrier()
    pltpu.sync_copy(agg_sh.at[pl.ds(s * N_SUB, N_SUB)],
                    out_hbm.at[c, pl.ds(s * N_SUB, N_SUB)])

    @pl.when(s == 0)
    def _():
        pltpu.sync_copy(agg_sh.at[pl.ds(NS * N_SUB, N_REM)],
                        out_hbm.at[c, pl.ds(NS * N_SUB, N_REM)])


def _sc_aggregate(x, row3, col3, zeros_block):
    mesh = plsc.VectorSubcoreMesh(core_axis_name="c", subcore_axis_name="s",
                                  num_cores=NC, num_subcores=NS)
    kern = pl.kernel(
        _sc_agg_body,
        out_type=jax.ShapeDtypeStruct((NC, N, D), jnp.float32),
        mesh=mesh,
        scratch_types=[
            pltpu.VMEM((EPW, CHUNK), jnp.int32),
            pltpu.VMEM((2 * NBANK, CHUNK), jnp.int32),
            pltpu.VMEM((CHUNK, D), jnp.float32),
            pltpu.VMEM((CHUNK, D), jnp.float32),
            pltpu.VMEM_SHARED((N_ACC, D), jnp.float32),
            pltpu.SemaphoreType.DMA,
            pltpu.SemaphoreType.DMA,
            pltpu.SemaphoreType.DMA,
            pltpu.SemaphoreType.DMA,
        ],
    )
    return kern(x, row3, col3, zeros_block)


def _mlp_body(eps_ref, x_ref, a0_ref, a1_ref, w1_ref, b1_ref, g1_ref,
              be1_ref, w2_ref, b2_ref, g2_ref, be2_ref, o_ref):
    h = x_ref[...] * (1.0 + eps_ref[0]) + a0_ref[...] + a1_ref[...]
    h = jnp.dot(h, w1_ref[...], preferred_element_type=jnp.float32)
    h = h + b1_ref[...]
    m = jnp.mean(h, axis=0, keepdims=True)
    hc = h - m
    v = jnp.mean(hc * hc, axis=0, keepdims=True)
    h = hc * lax.rsqrt(v + BN_EPS) * g1_ref[...] + be1_ref[...]
    h = jnp.maximum(h, 0.0)
    h = jnp.dot(h, w2_ref[...], preferred_element_type=jnp.float32)
    h = h + b2_ref[...]
    m2 = jnp.mean(h, axis=0, keepdims=True)
    hc2 = h - m2
    v2 = jnp.mean(hc2 * hc2, axis=0, keepdims=True)
    o_ref[...] = hc2 * lax.rsqrt(v2 + BN_EPS) * g2_ref[...] + be2_ref[...]


def _mlp(eps, x, a0, a1, W1, b1, g1, be1, W2, b2, g2, be2):
    smem_spec = pl.BlockSpec(memory_space=pltpu.SMEM)
    vmem_spec = pl.BlockSpec(memory_space=pltpu.VMEM)
    return pl.pallas_call(
        _mlp_body,
        out_shape=jax.ShapeDtypeStruct((N, D), jnp.float32),
        in_specs=[smem_spec] + [vmem_spec] * 11,
        out_specs=vmem_spec,
    )(eps, x, a0, a1, W1, b1, g1, be1, W2, b2, g2, be2)


@jax.jit
def kernel(x, edge_index, W1, b1, g1, be1, W2, b2, g2, be2, eps):
    row3 = edge_index[0].reshape(NW, EPW, CHUNK)
    col3 = edge_index[1].reshape(NW, EPW, CHUNK)
    zeros_block = jnp.zeros((N_SUB, D), jnp.float32)
    parts = _sc_aggregate(x, row3, col3, zeros_block)
    return _mlp(eps, x, parts[0], parts[1],
                W1, b1.reshape(1, D), g1.reshape(1, D), be1.reshape(1, D),
                W2, b2.reshape(1, D), g2.reshape(1, D), be2.reshape(1, D))


# speedup vs baseline: 2.2196x; 1.0440x over previous
import jax
import jax.numpy as jnp
from jax import lax
from jax.experimental import pallas as pl
from jax.experimental.pallas import tpu as pltpu
from jax.experimental.pallas import tpu_sc as plsc

N = 10000
E = 320000
D = 128
BN_EPS = 1e-5

NC = 2
NS = 16
NW = NC * NS

CHUNK = 125
EPW = E // (NW * CHUNK)
N_SUB = 624
N_REM = N - NS * N_SUB
N_ACC = N


NBANK = 8
NBANKS = EPW // NBANK


def _sc_agg_body(x_hbm, row_hbm, col_hbm, z_hbm, out_hbm,
                 ridx, cidx, buf0, buf1, agg_sh,
                 sem_r, sem_c, sem0, sem1):
    c = lax.axis_index("c")
    s = lax.axis_index("s")
    w = c * NS + s

    cp_r = pltpu.make_async_copy(row_hbm.at[w], ridx, sem_r)
    cp_r.start()
    pltpu.sync_copy(col_hbm.at[w, pl.ds(0, NBANK)], cidx.at[pl.ds(0, NBANK)])
    pltpu.make_async_copy(col_hbm.at[w, pl.ds(NBANK, NBANK)],
                          cidx.at[pl.ds(NBANK, NBANK)], sem_c).start()

    pltpu.sync_copy(z_hbm, agg_sh.at[pl.ds(s * N_SUB, N_SUB)])

    @pl.when(s == 0)
    def _():
        pltpu.sync_copy(z_hbm.at[pl.ds(0, N_REM)],
                        agg_sh.at[pl.ds(NS * N_SUB, N_REM)])

    cp_r.wait()
    plsc.subcore_barrier()

    def gather_start(j, buf, sem):
        pltpu.make_async_copy(x_hbm.at[ridx.at[j]], buf, sem).start()

    def gather_wait(j, buf, sem):
        pltpu.make_async_copy(x_hbm.at[ridx.at[j]], buf, sem).wait()

    bufs = (buf0, buf1)
    sems = (sem0, sem1)
    gather_start(0, buf0, sem0)
    gather_start(1, buf1, sem1)

    @pl.loop(0, NBANKS)
    def _(b):
        pb = lax.rem(b, 2)

        @pl.when(b > 0)
        def _():
            pltpu.make_async_copy(col_hbm.at[w, pl.ds(b * NBANK, NBANK)],
                                  cidx.at[pl.ds(pb * NBANK, NBANK)],
                                  sem_c).wait()

        @pl.when(b < NBANKS - 1)
        def _():
            pltpu.make_async_copy(
                col_hbm.at[w, pl.ds((b + 1) * NBANK, NBANK)],
                cidx.at[pl.ds((1 - pb) * NBANK, NBANK)], sem_c).start()

        for k in range(NBANK):
            j = b * NBANK + k
            p = k & 1
            gather_wait(j, bufs[p], sems[p])
            pltpu.sync_copy(bufs[p], agg_sh.at[cidx.at[pb * NBANK + k]],
                            add=True)

            @pl.when(j + 2 < EPW)
            def _():
                gather_start(j + 2, bufs[p], sems[p])

    plsc.subcore_barrier()
    pltpu.sync_copy(agg_sh.at[pl.ds(s * N_SUB, N_SUB)],
                    out_hbm.at[c, pl.ds(s * N_SUB, N_SUB)])

    @pl.when(s == 0)
    def _():
        pltpu.sync_copy(agg_sh.at[pl.ds(NS * N_SUB, N_REM)],
                        out_hbm.at[c, pl.ds(NS * N_SUB, N_REM)])


def _sc_aggregate(x, row3, col3, zeros_block):
    mesh = plsc.VectorSubcoreMesh(core_axis_name="c", subcore_axis_name="s",
                                  num_cores=NC, num_subcores=NS)
    kern = pl.kernel(
        _sc_agg_body,
        out_type=jax.ShapeDtypeStruct((NC, N, D), jnp.float32),
        mesh=mesh,
        scratch_types=[
            pltpu.VMEM((EPW, CHUNK), jnp.int32),
            pltpu.VMEM((2 * NBANK, CHUNK), jnp.int32),
            pltpu.VMEM((CHUNK, D), jnp.float32),
            pltpu.VMEM((CHUNK, D), jnp.float32),
            pltpu.VMEM_SHARED((N_ACC, D), jnp.float32),
            pltpu.SemaphoreType.DMA,
            pltpu.SemaphoreType.DMA,
            pltpu.SemaphoreType.DMA,
            pltpu.SemaphoreType.DMA,
        ],
    )
    return kern(x, row3, col3, zeros_block)


def _mlp_body(eps_ref, x_ref, a_ref, w1_ref, b1_ref, g1_ref,
              be1_ref, w2_ref, b2_ref, g2_ref, be2_ref, o_ref):
    h = x_ref[...] * (1.0 + eps_ref[0]) + a_ref[0] + a_ref[1]
    h = jnp.dot(h, w1_ref[...], preferred_element_type=jnp.float32)
    h = h + b1_ref[...]
    m = jnp.mean(h, axis=0, keepdims=True)
    hc = h - m
    v = jnp.mean(hc * hc, axis=0, keepdims=True)
    h = hc * lax.rsqrt(v + BN_EPS) * g1_ref[...] + be1_ref[...]
    h = jnp.maximum(h, 0.0)
    h = jnp.dot(h, w2_ref[...], preferred_element_type=jnp.float32)
    h = h + b2_ref[...]
    m2 = jnp.mean(h, axis=0, keepdims=True)
    hc2 = h - m2
    v2 = jnp.mean(hc2 * hc2, axis=0, keepdims=True)
    o_ref[...] = hc2 * lax.rsqrt(v2 + BN_EPS) * g2_ref[...] + be2_ref[...]


def _mlp(eps, x, parts, W1, b1, g1, be1, W2, b2, g2, be2):
    smem_spec = pl.BlockSpec(memory_space=pltpu.SMEM)
    vmem_spec = pl.BlockSpec(memory_space=pltpu.VMEM)
    return pl.pallas_call(
        _mlp_body,
        out_shape=jax.ShapeDtypeStruct((N, D), jnp.float32),
        in_specs=[smem_spec] + [vmem_spec] * 10,
        out_specs=vmem_spec,
    )(eps, x, parts, W1, b1, g1, be1, W2, b2, g2, be2)


@jax.jit
def kernel(x, edge_index, W1, b1, g1, be1, W2, b2, g2, be2, eps):
    row3 = edge_index[0].reshape(NW, EPW, CHUNK)
    col3 = edge_index[1].reshape(NW, EPW, CHUNK)
    zeros_block = jnp.zeros((N_SUB, D), jnp.float32)
    parts = _sc_aggregate(x, row3, col3, zeros_block)
    return _mlp(eps, x, parts,
                W1, b1.reshape(1, D), g1.reshape(1, D), be1.reshape(1, D),
                W2, b2.reshape(1, D), g2.reshape(1, D), be2.reshape(1, D))


# speedup vs baseline: 2.2244x; 1.0022x over previous
import jax
import jax.numpy as jnp
from jax import lax
from jax.experimental import pallas as pl
from jax.experimental.pallas import tpu as pltpu
from jax.experimental.pallas import tpu_sc as plsc

N = 10000
E = 320000
D = 128
BN_EPS = 1e-5

NC = 2
NS = 16
NW = NC * NS

CHUNK = 125
EPW = E // (NW * CHUNK)
N_SUB = 624
N_REM = N - NS * N_SUB
N_ACC = N


NBANK = 8
NBANKS = EPW // NBANK


def _sc_agg_body(x_hbm, row_hbm, col_hbm, z_hbm, out_hbm,
                 ridx, cidx, buf0, buf1, agg_sh,
                 sem_r, sem_c, sem0, sem1):
    c = lax.axis_index("c")
    s = lax.axis_index("s")
    w = c * NS + s

    cp_r = pltpu.make_async_copy(row_hbm.at[w], ridx, sem_r)
    cp_r.start()
    pltpu.sync_copy(col_hbm.at[w, pl.ds(0, NBANK)], cidx.at[pl.ds(0, NBANK)])
    pltpu.make_async_copy(col_hbm.at[w, pl.ds(NBANK, NBANK)],
                          cidx.at[pl.ds(NBANK, NBANK)], sem_c).start()

    pltpu.sync_copy(z_hbm, agg_sh.at[pl.ds(s * N_SUB, N_SUB)])

    @pl.when(s == 0)
    def _():
        pltpu.sync_copy(z_hbm.at[pl.ds(0, N_REM)],
                        agg_sh.at[pl.ds(NS * N_SUB, N_REM)])

    cp_r.wait()
    plsc.subcore_barrier()

    HALF = 64

    def gather_start(j, buf, sem):
        pltpu.make_async_copy(x_hbm.at[ridx.at[j, pl.ds(0, HALF)]],
                              buf.at[pl.ds(0, HALF)], sem).start()
        pltpu.make_async_copy(x_hbm.at[ridx.at[j, pl.ds(HALF, CHUNK - HALF)]],
                              buf.at[pl.ds(HALF, CHUNK - HALF)], sem).start()

    def gather_wait(j, buf, sem):
        pltpu.make_async_copy(x_hbm.at[ridx.at[j, pl.ds(0, HALF)]],
                              buf.at[pl.ds(0, HALF)], sem).wait()
        pltpu.make_async_copy(x_hbm.at[ridx.at[j, pl.ds(HALF, CHUNK - HALF)]],
                              buf.at[pl.ds(HALF, CHUNK - HALF)], sem).wait()

    bufs = (buf0, buf1)
    sems = (sem0, sem1)
    gather_start(0, buf0, sem0)
    gather_start(1, buf1, sem1)

    @pl.loop(0, NBANKS)
    def _(b):
        pb = lax.rem(b, 2)

        @pl.when(b > 0)
        def _():
            pltpu.make_async_copy(col_hbm.at[w, pl.ds(b * NBANK, NBANK)],
                                  cidx.at[pl.ds(pb * NBANK, NBANK)],
                                  sem_c).wait()

        @pl.when(b < NBANKS - 1)
        def _():
            pltpu.make_async_copy(
                col_hbm.at[w, pl.ds((b + 1) * NBANK, NBANK)],
                cidx.at[pl.ds((1 - pb) * NBANK, NBANK)], sem_c).start()

        for k in range(NBANK):
            j = b * NBANK + k
            p = k & 1
            gather_wait(j, bufs[p], sems[p])
            pltpu.sync_copy(bufs[p], agg_sh.at[cidx.at[pb * NBANK + k]],
                            add=True)

            @pl.when(j + 2 < EPW)
            def _():
                gather_start(j + 2, bufs[p], sems[p])

    plsc.subcore_barrier()
    pltpu.sync_copy(agg_sh.at[pl.ds(s * N_SUB, N_SUB)],
                    out_hbm.at[c, pl.ds(s * N_SUB, N_SUB)])

    @pl.when(s == 0)
    def _():
        pltpu.sync_copy(agg_sh.at[pl.ds(NS * N_SUB, N_REM)],
                        out_hbm.at[c, pl.ds(NS * N_SUB, N_REM)])


def _sc_aggregate(x, row3, col3, zeros_block):
    mesh = plsc.VectorSubcoreMesh(core_axis_name="c", subcore_axis_name="s",
                                  num_cores=NC, num_subcores=NS)
    kern = pl.kernel(
        _sc_agg_body,
        out_type=jax.ShapeDtypeStruct((NC, N, D), jnp.float32),
        mesh=mesh,
        scratch_types=[
            pltpu.VMEM((EPW, CHUNK), jnp.int32),
            pltpu.VMEM((2 * NBANK, CHUNK), jnp.int32),
            pltpu.VMEM((CHUNK, D), jnp.float32),
            pltpu.VMEM((CHUNK, D), jnp.float32),
            pltpu.VMEM_SHARED((N_ACC, D), jnp.float32),
            pltpu.SemaphoreType.DMA,
            pltpu.SemaphoreType.DMA,
            pltpu.SemaphoreType.DMA,
            pltpu.SemaphoreType.DMA,
        ],
    )
    return kern(x, row3, col3, zeros_block)


def _mlp_body(eps_ref, x_ref, a_ref, w1_ref, b1_ref, g1_ref,
              be1_ref, w2_ref, b2_ref, g2_ref, be2_ref, o_ref):
    h = x_ref[...] * (1.0 + eps_ref[0]) + a_ref[0] + a_ref[1]
    h = jnp.dot(h, w1_ref[...], preferred_element_type=jnp.float32)
    h = h + b1_ref[...]
    m = jnp.mean(h, axis=0, keepdims=True)
    hc = h - m
    v = jnp.mean(hc * hc, axis=0, keepdims=True)
    h = hc * lax.rsqrt(v + BN_EPS) * g1_ref[...] + be1_ref[...]
    h = jnp.maximum(h, 0.0)
    h = jnp.dot(h, w2_ref[...], preferred_element_type=jnp.float32)
    h = h + b2_ref[...]
    m2 = jnp.mean(h, axis=0, keepdims=True)
    hc2 = h - m2
    v2 = jnp.mean(hc2 * hc2, axis=0, keepdims=True)
    o_ref[...] = hc2 * lax.rsqrt(v2 + BN_EPS) * g2_ref[...] + be2_ref[...]


def _mlp(eps, x, parts, W1, b1, g1, be1, W2, b2, g2, be2):
    smem_spec = pl.BlockSpec(memory_space=pltpu.SMEM)
    vmem_spec = pl.BlockSpec(memory_space=pltpu.VMEM)
    return pl.pallas_call(
        _mlp_body,
        out_shape=jax.ShapeDtypeStruct((N, D), jnp.float32),
        in_specs=[smem_spec] + [vmem_spec] * 10,
        out_specs=vmem_spec,
    )(eps, x, parts, W1, b1, g1, be1, W2, b2, g2, be2)


@jax.jit
def kernel(x, edge_index, W1, b1, g1, be1, W2, b2, g2, be2, eps):
    row3 = edge_index[0].reshape(NW, EPW, CHUNK)
    col3 = edge_index[1].reshape(NW, EPW, CHUNK)
    zeros_block = jnp.zeros((N_SUB, D), jnp.float32)
    parts = _sc_aggregate(x, row3, col3, zeros_block)
    return _mlp(eps, x, parts,
                W1, b1.reshape(1, D), g1.reshape(1, D), be1.reshape(1, D),
                W2, b2.reshape(1, D), g2.reshape(1, D), be2.reshape(1, D))
